# baseline (speedup 1.0000x reference)
import numpy as np

import jax
import jax.numpy as jnp
from jax import lax
from jax.experimental import pallas as pl
from jax.experimental.pallas import tpu as pltpu
from jax.experimental.pallas import tpu_sc as plsc

B = 128
V = 100000
K = 256
CAP1 = 4096
CAP2 = 1024
NW = 32
ROWS_PER_W = B // NW
IGNORED = -3000.0
M_ALL = np.uint32(0xFFFFFFFF)
M_SIGN = np.uint32(0x80000000)


def _keys_of(x):
    u = lax.bitcast_convert_type(x, jnp.uint32)
    return jnp.where((u >> 31) == jnp.uint32(1), u ^ M_ALL, u ^ M_SIGN)


def _select_bucket(hist_refs, kval, lane):
    def chunk_body(jj, carry):
        above, b_sel, g_sel = carry
        base = (15 - jj) * 16
        acc = jnp.zeros((16,), jnp.int32)
        for h in hist_refs:
            for l in range(16):
                acc = acc + h[pl.ds(l * 256 + base, 16)]
        total = jnp.sum(acc)
        pref = plsc.cumsum(acc)
        g = above + total - pref
        m = (g < kval) & (g + acc >= kval)
        b_sel = b_sel + jnp.sum(jnp.where(m, base + lane, 0))
        g_sel = g_sel + jnp.sum(jnp.where(m, g, 0))
        return (above + total, b_sel, g_sel)

    init = (jnp.int32(0), jnp.int32(0), jnp.int32(0))
    _, b_sel, g_sel = lax.fori_loop(0, 16, chunk_body, init)
    return b_sel, g_sel


GRP = 10
NGRP = V // (16 * GRP)
NBANK = 4


def _sc_select_body(logits_hbm, outk_hbm, outi_hbm,
                    row_v, candk_v, candi_v, h0, h1, h2, h3, outk_v, outi_v,
                    gmax_s):
    wid = lax.axis_index("s") * 2 + lax.axis_index("c")
    lane = lax.iota(jnp.int32, 16)
    ones = jnp.ones((16,), jnp.int32)
    zeros_i = jnp.zeros((16,), jnp.int32)
    zeros_u = jnp.zeros((16,), jnp.uint32)
    banks = (h0, h1, h2, h3)
    lane256 = lane * 256

    def zero_all_hists(i, _):
        for h in banks:
            h[pl.ds(i * 16, 16)] = zeros_i
        return 0

    def zero_hist(i, _):
        h0[pl.ds(i * 16, 16)] = zeros_i
        return 0

    def row_body(r, _):
        row = wid * ROWS_PER_W + r
        pltpu.sync_copy(logits_hbm.at[row], row_v)

        lax.fori_loop(0, 256, zero_all_hists, 0)

        def pass_a(g, _):
            base = g * (16 * GRP)
            mx = jnp.full((16,), -2147483648, jnp.int32)
            for j in range(GRP):
                key = _keys_of(row_v[pl.ds(base + j * 16, 16)])
                mx = jnp.maximum(mx, lax.bitcast_convert_type(key ^ M_SIGN, jnp.int32))
                bin1 = (key >> 24).astype(jnp.int32)
                plsc.addupdate_scatter(banks[j % NBANK], [lane256 + bin1], ones)
            gmax_s[g] = jnp.max(mx)
            return 0

        lax.fori_loop(0, NGRP, pass_a, 0)
        b1, g1 = _select_bucket(banks, jnp.int32(K), lane)
        b1u = b1.astype(jnp.uint32)
        thr = lax.bitcast_convert_type((b1u << 24) ^ M_SIGN, jnp.int32)

        def pass_b(g, ptr):
            def do(ptr):
                base = g * (16 * GRP)
                for j in range(GRP):
                    key = _keys_of(row_v[pl.ds(base + j * 16, 16)])
                    m = (key >> 24) >= b1u
                    cnt = jnp.sum(m.astype(jnp.int32))
                    sp = jnp.minimum(ptr, CAP1)
                    plsc.store_compressed(candk_v.at[pl.ds(sp, 16)], key, mask=m)
                    plsc.store_compressed(
                        candi_v.at[pl.ds(sp, 16)], base + j * 16 + lane, mask=m)
                    ptr = ptr + cnt
                return ptr

            return lax.cond(gmax_s[g] >= thr, do, lambda p: p, ptr)

        n1 = lax.fori_loop(0, NGRP, pass_b, jnp.int32(0))
        n1 = jnp.minimum(n1, CAP1)
        ntrip = (n1 + 15) // 16

        lax.fori_loop(0, 256, zero_hist, 0)

        def pass_l2(i, _):
            kv = candk_v[pl.ds(i * 16, 16)]
            valid = lane < (n1 - i * 16)
            mb = valid & ((kv >> 24) == b1u)
            bin2 = ((kv >> 16) & jnp.uint32(0xFF)).astype(jnp.int32)
            plsc.addupdate_scatter(h0, [lane256 + bin2], ones, mask=mb)
            return 0

        lax.fori_loop(0, ntrip, pass_l2, 0)
        b2, _ = _select_bucket((h0,), jnp.int32(K) - g1, lane)
        t16 = (b1u << 8) | b2.astype(jnp.uint32)

        def zero_out(i, _):
            outk_v[pl.ds(i * 16, 16)] = zeros_u
            outi_v[pl.ds(i * 16, 16)] = zeros_i
            return 0

        lax.fori_loop(0, CAP2 // 16 + 1, zero_out, 0)

        def make_pass_c(cmp_eq):
            def pass_c(i, ptr):
                kv = candk_v[pl.ds(i * 16, 16)]
                iv = candi_v[pl.ds(i * 16, 16)]
                valid = lane < (n1 - i * 16)
                hi = kv >> 16
                m = valid & ((hi == t16) if cmp_eq else (hi > t16))
                cnt = jnp.sum(m.astype(jnp.int32))
                sp = jnp.minimum(ptr, CAP2)
                plsc.store_compressed(outk_v.at[pl.ds(sp, 16)], kv, mask=m)
                plsc.store_compressed(outi_v.at[pl.ds(sp, 16)], iv, mask=m)
                return ptr + cnt

            return pass_c

        ng = lax.fori_loop(0, ntrip, make_pass_c(False), jnp.int32(0))
        lax.fori_loop(0, ntrip, make_pass_c(True), ng)

        pltpu.sync_copy(outk_v.at[pl.ds(0, CAP2)], outk_hbm.at[row])
        pltpu.sync_copy(outi_v.at[pl.ds(0, CAP2)], outi_hbm.at[row])
        return 0

    lax.fori_loop(0, ROWS_PER_W, row_body, 0)


def _tc_sample_body(keys_ref, idx_ref, tk_ref, tp_ref, temp_ref, rand_ref, out_ref):
    n = CAP2
    minint = jnp.int32(-2147483648)
    skey = lax.bitcast_convert_type(keys_ref[...], jnp.int32) ^ minint
    m = ~skey
    idx = idx_ref[...]
    col = lax.broadcasted_iota(jnp.int32, (B, n), 1)

    k = 2
    while k <= n:
        j = k // 2
        while j >= 1:
            lower = (col & j) == 0
            pm = jnp.where(lower, pltpu.roll(m, n - j, 1), pltpu.roll(m, j, 1))
            pidx = jnp.where(lower, pltpu.roll(idx, n - j, 1), pltpu.roll(idx, j, 1))
            precedes = (m < pm) | ((m == pm) & (idx < pidx))
            asc = (col & k) == 0
            keep_self = asc == (lower == precedes)
            m = jnp.where(keep_self, m, pm)
            idx = jnp.where(keep_self, idx, pidx)
            j //= 2
        k *= 2

    ki = (~m[:, :K]) ^ minint
    xbits = jnp.where(ki < 0, ki ^ minint, ~ki)
    vals = lax.bitcast_convert_type(xbits, jnp.float32)
    idx_s = idx[:, :K]
    colk = lax.broadcasted_iota(jnp.int32, (B, K), 1)

    tk = tk_ref[...]
    tp = tp_ref[...]
    temp = temp_ref[...]
    rand = rand_ref[...]

    vals = jnp.where(colk.astype(jnp.float32) >= tk, IGNORED, vals) / temp

    def softmax(v):
        mx = jnp.max(v, axis=1, keepdims=True)
        e = jnp.exp(v - mx)
        return e / jnp.sum(e, axis=1, keepdims=True)

    def cumsum(p):
        c = p
        d = 1
        while d < K:
            c = c + jnp.where(colk >= d, pltpu.roll(c, d, 1), 0.0)
            d *= 2
        return c

    csum = cumsum(softmax(vals))
    tp_eff = jnp.maximum(jnp.min(csum), tp)
    mask2 = (csum > tp_eff) & (colk >= 1)
    vals = jnp.where(mask2, IGNORED, vals)
    csum2 = cumsum(softmax(vals))
    counts = jnp.sum((rand > csum2).astype(jnp.int32), axis=1, keepdims=True)
    counts = jnp.minimum(counts, K - 1)
    out_ref[...] = jnp.sum(jnp.where(colk == counts, idx_s, 0), axis=1, keepdims=True)


def kernel(token_logits, sampling_params):
    mesh = plsc.VectorSubcoreMesh(core_axis_name="c", subcore_axis_name="s")
    sc_select = pl.kernel(
        _sc_select_body,
        out_type=(
            jax.ShapeDtypeStruct((B, CAP2), jnp.uint32),
            jax.ShapeDtypeStruct((B, CAP2), jnp.int32),
        ),
        mesh=mesh,
        compiler_params=pltpu.CompilerParams(needs_layout_passes=False),
        scratch_types=[
            pltpu.VMEM((V,), jnp.float32),
            pltpu.VMEM((CAP1 + 16,), jnp.uint32),
            pltpu.VMEM((CAP1 + 16,), jnp.int32),
            pltpu.VMEM((4096,), jnp.int32),
            pltpu.VMEM((4096,), jnp.int32),
            pltpu.VMEM((4096,), jnp.int32),
            pltpu.VMEM((4096,), jnp.int32),
            pltpu.VMEM((CAP2 + 16,), jnp.uint32),
            pltpu.VMEM((CAP2 + 16,), jnp.int32),
            pltpu.SMEM((NGRP,), jnp.int32),
        ],
    )
    keys, idx = sc_select(token_logits)

    tk = sampling_params[:, 0:1]
    tp = sampling_params[:, 1:2]
    temp = sampling_params[:, 2:3]
    rand = jax.random.uniform(jax.random.key(1234), (B, 1), dtype=jnp.float32)

    tokens = pl.pallas_call(
        _tc_sample_body,
        out_shape=jax.ShapeDtypeStruct((B, 1), jnp.int32),
    )(keys, idx, tk, tp, temp, rand)
    return tokens.reshape(-1)


# speedup vs baseline: 1.3791x; 1.3791x over previous
import numpy as np

import jax
import jax.numpy as jnp
from jax import lax
from jax.experimental import pallas as pl
from jax.experimental.pallas import tpu as pltpu
from jax.experimental.pallas import tpu_sc as plsc

B = 128
V = 100000
K = 256
CAP1 = 4096
CAP2 = 1024
NW = 32
ROWS_PER_W = B // NW
IGNORED = -3000.0
M_ALL = np.uint32(0xFFFFFFFF)
M_SIGN = np.uint32(0x80000000)


def _keys_of(x):
    u = lax.bitcast_convert_type(x, jnp.uint32)
    return jnp.where((u >> 31) == jnp.uint32(1), u ^ M_ALL, u ^ M_SIGN)


def _select_bucket(hist_refs, kval, lane):
    def chunk_body(jj, carry):
        above, b_sel, g_sel = carry
        base = (15 - jj) * 16
        acc = jnp.zeros((16,), jnp.int32)
        for h in hist_refs:
            for l in range(16):
                acc = acc + h[pl.ds(l * 256 + base, 16)]
        total = jnp.sum(acc)
        pref = plsc.cumsum(acc)
        g = above + total - pref
        m = (g < kval) & (g + acc >= kval)
        b_sel = b_sel + jnp.sum(jnp.where(m, base + lane, 0))
        g_sel = g_sel + jnp.sum(jnp.where(m, g, 0))
        return (above + total, b_sel, g_sel)

    init = (jnp.int32(0), jnp.int32(0), jnp.int32(0))
    _, b_sel, g_sel = lax.fori_loop(0, 16, chunk_body, init)
    return b_sel, g_sel


GRP = 10
NGRP = V // (16 * GRP)
NBANK = 4


def _sc_select_body(logits_hbm, outk_hbm, outi_hbm,
                    row_v, candk_v, candi_v, h0, h1, outk_v, outi_v):
    wid = lax.axis_index("s") * 2 + lax.axis_index("c")
    lane = lax.iota(jnp.int32, 16)
    ones = jnp.ones((16,), jnp.int32)
    zeros_i = jnp.zeros((16,), jnp.int32)
    zeros_u = jnp.zeros((16,), jnp.uint32)
    banks = (h0, h1)
    lane256 = lane * 256

    def zero_all_hists(i, _):
        for h in banks:
            h[pl.ds(i * 16, 16)] = zeros_i
        return 0

    def zero_hist(i, _):
        h0[pl.ds(i * 16, 16)] = zeros_i
        return 0

    def row_body(r, _):
        row = wid * ROWS_PER_W + r
        pltpu.sync_copy(logits_hbm.at[row], row_v)

        lax.fori_loop(0, 256, zero_all_hists, 0)

        def pass_a(i, _):
            base = i * 32
            for j in range(2):
                key = _keys_of(row_v[pl.ds(base + j * 16, 16)])
                bin1 = (key >> 24).astype(jnp.int32)
                plsc.addupdate_scatter(banks[j], [lane256 + bin1], ones)
            return 0

        lax.fori_loop(0, V // 32, pass_a, 0)
        b1, g1 = _select_bucket(banks, jnp.int32(K), lane)
        b1u = b1.astype(jnp.uint32)

        def pass_b(i, ptr):
            key = _keys_of(row_v[pl.ds(i * 16, 16)])
            m = (key >> 24) >= b1u
            cnt = jnp.sum(m.astype(jnp.int32))
            sp = jnp.minimum(ptr, CAP1)
            plsc.store_compressed(candk_v.at[pl.ds(sp, 16)], key, mask=m)
            plsc.store_compressed(candi_v.at[pl.ds(sp, 16)], i * 16 + lane, mask=m)
            return ptr + cnt

        n1 = lax.fori_loop(0, V // 16, pass_b, jnp.int32(0))
        n1 = jnp.minimum(n1, CAP1)
        ntrip = (n1 + 15) // 16

        lax.fori_loop(0, 256, zero_hist, 0)

        def pass_l2(i, _):
            kv = candk_v[pl.ds(i * 16, 16)]
            valid = lane < (n1 - i * 16)
            mb = valid & ((kv >> 24) == b1u)
            bin2 = ((kv >> 16) & jnp.uint32(0xFF)).astype(jnp.int32)
            plsc.addupdate_scatter(h0, [lane256 + bin2], ones, mask=mb)
            return 0

        lax.fori_loop(0, ntrip, pass_l2, 0)
        b2, _ = _select_bucket((h0,), jnp.int32(K) - g1, lane)
        t16 = (b1u << 8) | b2.astype(jnp.uint32)

        def zero_out(i, _):
            outk_v[pl.ds(i * 16, 16)] = zeros_u
            outi_v[pl.ds(i * 16, 16)] = zeros_i
            return 0

        lax.fori_loop(0, CAP2 // 16 + 1, zero_out, 0)

        def make_pass_c(cmp_eq):
            def pass_c(i, ptr):
                kv = candk_v[pl.ds(i * 16, 16)]
                iv = candi_v[pl.ds(i * 16, 16)]
                valid = lane < (n1 - i * 16)
                hi = kv >> 16
                m = valid & ((hi == t16) if cmp_eq else (hi > t16))
                cnt = jnp.sum(m.astype(jnp.int32))
                sp = jnp.minimum(ptr, CAP2)
                plsc.store_compressed(outk_v.at[pl.ds(sp, 16)], kv, mask=m)
                plsc.store_compressed(outi_v.at[pl.ds(sp, 16)], iv, mask=m)
                return ptr + cnt

            return pass_c

        ng = lax.fori_loop(0, ntrip, make_pass_c(False), jnp.int32(0))
        lax.fori_loop(0, ntrip, make_pass_c(True), ng)

        pltpu.sync_copy(outk_v.at[pl.ds(0, CAP2)], outk_hbm.at[row])
        pltpu.sync_copy(outi_v.at[pl.ds(0, CAP2)], outi_hbm.at[row])
        return 0

    lax.fori_loop(0, ROWS_PER_W, row_body, 0)


def _tc_sample_body(keys_ref, idx_ref, tk_ref, tp_ref, temp_ref, rand_ref, out_ref):
    n = CAP2
    minint = jnp.int32(-2147483648)
    skey = lax.bitcast_convert_type(keys_ref[...], jnp.int32) ^ minint
    m = ~skey
    idx = idx_ref[...]
    col = lax.broadcasted_iota(jnp.int32, (B, n), 1)

    k = 2
    while k <= n:
        j = k // 2
        while j >= 1:
            lower = (col & j) == 0
            pm = jnp.where(lower, pltpu.roll(m, n - j, 1), pltpu.roll(m, j, 1))
            pidx = jnp.where(lower, pltpu.roll(idx, n - j, 1), pltpu.roll(idx, j, 1))
            precedes = (m < pm) | ((m == pm) & (idx < pidx))
            asc = (col & k) == 0
            keep_self = asc == (lower == precedes)
            m = jnp.where(keep_self, m, pm)
            idx = jnp.where(keep_self, idx, pidx)
            j //= 2
        k *= 2

    ki = (~m[:, :K]) ^ minint
    xbits = jnp.where(ki < 0, ki ^ minint, ~ki)
    vals = lax.bitcast_convert_type(xbits, jnp.float32)
    idx_s = idx[:, :K]
    colk = lax.broadcasted_iota(jnp.int32, (B, K), 1)

    tk = tk_ref[...]
    tp = tp_ref[...]
    temp = temp_ref[...]
    rand = rand_ref[...]

    vals = jnp.where(colk.astype(jnp.float32) >= tk, IGNORED, vals) / temp

    def softmax(v):
        mx = jnp.max(v, axis=1, keepdims=True)
        e = jnp.exp(v - mx)
        return e / jnp.sum(e, axis=1, keepdims=True)

    def cumsum(p):
        c = p
        d = 1
        while d < K:
            c = c + jnp.where(colk >= d, pltpu.roll(c, d, 1), 0.0)
            d *= 2
        return c

    csum = cumsum(softmax(vals))
    tp_eff = jnp.maximum(jnp.min(csum), tp)
    mask2 = (csum > tp_eff) & (colk >= 1)
    vals = jnp.where(mask2, IGNORED, vals)
    csum2 = cumsum(softmax(vals))
    counts = jnp.sum((rand > csum2).astype(jnp.int32), axis=1, keepdims=True)
    counts = jnp.minimum(counts, K - 1)
    out_ref[...] = jnp.sum(jnp.where(colk == counts, idx_s, 0), axis=1, keepdims=True)


def kernel(token_logits, sampling_params):
    mesh = plsc.VectorSubcoreMesh(core_axis_name="c", subcore_axis_name="s")
    sc_select = pl.kernel(
        _sc_select_body,
        out_type=(
            jax.ShapeDtypeStruct((B, CAP2), jnp.uint32),
            jax.ShapeDtypeStruct((B, CAP2), jnp.int32),
        ),
        mesh=mesh,
        compiler_params=pltpu.CompilerParams(needs_layout_passes=False),
        scratch_types=[
            pltpu.VMEM((V,), jnp.float32),
            pltpu.VMEM((CAP1 + 16,), jnp.uint32),
            pltpu.VMEM((CAP1 + 16,), jnp.int32),
            pltpu.VMEM((4096,), jnp.int32),
            pltpu.VMEM((4096,), jnp.int32),
            pltpu.VMEM((CAP2 + 16,), jnp.uint32),
            pltpu.VMEM((CAP2 + 16,), jnp.int32),
        ],
    )
    keys, idx = sc_select(token_logits)

    tk = sampling_params[:, 0:1]
    tp = sampling_params[:, 1:2]
    temp = sampling_params[:, 2:3]
    rand = jax.random.uniform(jax.random.key(1234), (B, 1), dtype=jnp.float32)

    tokens = pl.pallas_call(
        _tc_sample_body,
        out_shape=jax.ShapeDtypeStruct((B, 1), jnp.int32),
    )(keys, idx, tk, tp, temp, rand)
    return tokens.reshape(-1)


# speedup vs baseline: 2.3542x; 1.7071x over previous
import numpy as np

import jax
import jax.numpy as jnp
from jax import lax
from jax.experimental import pallas as pl
from jax.experimental.pallas import tpu as pltpu
from jax.experimental.pallas import tpu_sc as plsc

B = 128
V = 100000
K = 256
CAP1 = 4096
CAP2 = 1024
NW = 32
ROWS_PER_W = B // NW
IGNORED = -3000.0
M_ALL = np.uint32(0xFFFFFFFF)
M_SIGN = np.uint32(0x80000000)


def _keys_of(x):
    u = lax.bitcast_convert_type(x, jnp.uint32)
    return jnp.where((u >> 31) == jnp.uint32(1), u ^ M_ALL, u ^ M_SIGN)


def _select_bucket(hist_v, kval, lane):
    def chunk_body(jj, carry):
        above, b_sel, g_sel = carry
        base = (15 - jj) * 16
        acc = jnp.zeros((16,), jnp.int32)
        for l in range(16):
            acc = acc + plsc.load_gather(hist_v, [(base + lane) * 16 + l])
        total = jnp.sum(acc)
        pref = plsc.cumsum(acc)
        g = above + total - pref
        m = (g < kval) & (g + acc >= kval)
        b_sel = b_sel + jnp.sum(jnp.where(m, base + lane, 0))
        g_sel = g_sel + jnp.sum(jnp.where(m, g, 0))
        return (above + total, b_sel, g_sel)

    init = (jnp.int32(0), jnp.int32(0), jnp.int32(0))
    _, b_sel, g_sel = lax.fori_loop(0, 16, chunk_body, init)
    return b_sel, g_sel


def _sc_select_body(logits_hbm, outk_hbm, outi_hbm,
                    row_v, candk_v, candi_v, h0, outk_v, outi_v):
    wid = lax.axis_index("s") * 2 + lax.axis_index("c")
    lane = lax.iota(jnp.int32, 16)
    ones = jnp.ones((16,), jnp.int32)
    zeros_i = jnp.zeros((16,), jnp.int32)
    zeros_u = jnp.zeros((16,), jnp.uint32)

    def zero_hist(i, _):
        h0[pl.ds(i * 16, 16)] = zeros_i
        return 0

    def _popcnt(m):
        return plsc.all_reduce_population_count(m)[0]

    def row_body(r, _):
        row = wid * ROWS_PER_W + r
        pltpu.sync_copy(logits_hbm.at[row], row_v)

        @plsc.parallel_loop(0, 256, unroll=4)
        def _(i):
            h0[pl.ds(i * 16, 16)] = zeros_i

        @plsc.parallel_loop(0, V // 16, unroll=4)
        def _(i):
            key = _keys_of(row_v[pl.ds(i * 16, 16)])
            bin1 = (key >> 24).astype(jnp.int32)
            plsc.addupdate_scatter(h0, [bin1 * 16 + lane], ones)
        b1, g1 = _select_bucket(h0, jnp.int32(K), lane)
        b1u = b1.astype(jnp.uint32)

        def pass_b(i, ptr):
            key = _keys_of(row_v[pl.ds(i * 16, 16)])
            m = (key >> 24) >= b1u
            cnt = _popcnt(m)
            sp = jnp.minimum(ptr, CAP1)
            plsc.store_compressed(candk_v.at[pl.ds(sp, 16)], key, mask=m)
            plsc.store_compressed(candi_v.at[pl.ds(sp, 16)], i * 16 + lane, mask=m)
            return ptr + cnt

        n1 = lax.fori_loop(0, V // 16, pass_b, jnp.int32(0))
        n1 = jnp.minimum(n1, CAP1)
        ntrip = (n1 + 15) // 16

        lax.fori_loop(0, 256, zero_hist, 0)

        def pass_l2(i, _):
            kv = candk_v[pl.ds(i * 16, 16)]
            valid = lane < (n1 - i * 16)
            mb = valid & ((kv >> 24) == b1u)
            bin2 = ((kv >> 16) & jnp.uint32(0xFF)).astype(jnp.int32)
            plsc.addupdate_scatter(h0, [bin2 * 16 + lane], ones, mask=mb)
            return 0

        lax.fori_loop(0, ntrip, pass_l2, 0)
        b2, _ = _select_bucket(h0, jnp.int32(K) - g1, lane)
        t16 = (b1u << 8) | b2.astype(jnp.uint32)

        def zero_out(i, _):
            outk_v[pl.ds(i * 16, 16)] = zeros_u
            outi_v[pl.ds(i * 16, 16)] = zeros_i
            return 0

        lax.fori_loop(0, CAP2 // 16 + 1, zero_out, 0)

        def make_pass_c(cmp_eq):
            def pass_c(i, ptr):
                kv = candk_v[pl.ds(i * 16, 16)]
                iv = candi_v[pl.ds(i * 16, 16)]
                valid = lane < (n1 - i * 16)
                hi = kv >> 16
                m = valid & ((hi == t16) if cmp_eq else (hi > t16))
                cnt = _popcnt(m)
                sp = jnp.minimum(ptr, CAP2)
                plsc.store_compressed(outk_v.at[pl.ds(sp, 16)], kv, mask=m)
                plsc.store_compressed(outi_v.at[pl.ds(sp, 16)], iv, mask=m)
                return ptr + cnt

            return pass_c

        ng = lax.fori_loop(0, ntrip, make_pass_c(False), jnp.int32(0))
        lax.fori_loop(0, ntrip, make_pass_c(True), ng)

        pltpu.sync_copy(outk_v.at[pl.ds(0, CAP2)], outk_hbm.at[row])
        pltpu.sync_copy(outi_v.at[pl.ds(0, CAP2)], outi_hbm.at[row])
        return 0

    lax.fori_loop(0, ROWS_PER_W, row_body, 0)


def _tc_sample_body(keys_ref, idx_ref, tk_ref, tp_ref, temp_ref, rand_ref, out_ref):
    n = CAP2
    minint = jnp.int32(-2147483648)
    skey = lax.bitcast_convert_type(keys_ref[...], jnp.int32) ^ minint
    m = ~skey
    idx = idx_ref[...]
    col = lax.broadcasted_iota(jnp.int32, (B, n), 1)

    k = 2
    while k <= n:
        j = k // 2
        while j >= 1:
            lower = (col & j) == 0
            pm = jnp.where(lower, pltpu.roll(m, n - j, 1), pltpu.roll(m, j, 1))
            pidx = jnp.where(lower, pltpu.roll(idx, n - j, 1), pltpu.roll(idx, j, 1))
            precedes = (m < pm) | ((m == pm) & (idx < pidx))
            asc = (col & k) == 0
            keep_self = asc == (lower == precedes)
            m = jnp.where(keep_self, m, pm)
            idx = jnp.where(keep_self, idx, pidx)
            j //= 2
        k *= 2

    ki = (~m[:, :K]) ^ minint
    xbits = jnp.where(ki < 0, ki ^ minint, ~ki)
    vals = lax.bitcast_convert_type(xbits, jnp.float32)
    idx_s = idx[:, :K]
    colk = lax.broadcasted_iota(jnp.int32, (B, K), 1)

    tk = tk_ref[...]
    tp = tp_ref[...]
    temp = temp_ref[...]
    rand = rand_ref[...]

    vals = jnp.where(colk.astype(jnp.float32) >= tk, IGNORED, vals) / temp

    def softmax(v):
        mx = jnp.max(v, axis=1, keepdims=True)
        e = jnp.exp(v - mx)
        return e / jnp.sum(e, axis=1, keepdims=True)

    def cumsum(p):
        c = p
        d = 1
        while d < K:
            c = c + jnp.where(colk >= d, pltpu.roll(c, d, 1), 0.0)
            d *= 2
        return c

    csum = cumsum(softmax(vals))
    tp_eff = jnp.maximum(jnp.min(csum), tp)
    mask2 = (csum > tp_eff) & (colk >= 1)
    vals = jnp.where(mask2, IGNORED, vals)
    csum2 = cumsum(softmax(vals))
    counts = jnp.sum((rand > csum2).astype(jnp.int32), axis=1, keepdims=True)
    counts = jnp.minimum(counts, K - 1)
    out_ref[...] = jnp.sum(jnp.where(colk == counts, idx_s, 0), axis=1, keepdims=True)


def kernel(token_logits, sampling_params):
    mesh = plsc.VectorSubcoreMesh(core_axis_name="c", subcore_axis_name="s")
    sc_select = pl.kernel(
        _sc_select_body,
        out_type=(
            jax.ShapeDtypeStruct((B, CAP2), jnp.uint32),
            jax.ShapeDtypeStruct((B, CAP2), jnp.int32),
        ),
        mesh=mesh,
        compiler_params=pltpu.CompilerParams(needs_layout_passes=False),
        scratch_types=[
            pltpu.VMEM((V,), jnp.float32),
            pltpu.VMEM((CAP1 + 16,), jnp.uint32),
            pltpu.VMEM((CAP1 + 16,), jnp.int32),
            pltpu.VMEM((4096,), jnp.int32),
            pltpu.VMEM((CAP2 + 16,), jnp.uint32),
            pltpu.VMEM((CAP2 + 16,), jnp.int32),
        ],
    )
    keys, idx = sc_select(token_logits)

    tk = sampling_params[:, 0:1]
    tp = sampling_params[:, 1:2]
    temp = sampling_params[:, 2:3]
    rand = jax.random.uniform(jax.random.key(1234), (B, 1), dtype=jnp.float32)

    tokens = pl.pallas_call(
        _tc_sample_body,
        out_shape=jax.ShapeDtypeStruct((B, 1), jnp.int32),
    )(keys, idx, tk, tp, temp, rand)
    return tokens.reshape(-1)


# speedup vs baseline: 3.3253x; 1.4125x over previous
import numpy as np

import jax
import jax.numpy as jnp
from jax import lax
from jax.experimental import pallas as pl
from jax.experimental.pallas import tpu as pltpu
from jax.experimental.pallas import tpu_sc as plsc

B = 128
V = 100000
K = 256
CAP1 = 4096
CAP2 = 1024
NW = 32
ROWS_PER_W = B // NW
IGNORED = -3000.0
M_ALL = np.uint32(0xFFFFFFFF)
M_SIGN = np.uint32(0x80000000)


def _keys_of(x):
    u = lax.bitcast_convert_type(x, jnp.uint32)
    return jnp.where((u >> 31) == jnp.uint32(1), u ^ M_ALL, u ^ M_SIGN)


def _select_bucket(hist_v, kval, lane):
    def chunk_body(jj, carry):
        above, b_sel, g_sel = carry
        base = (15 - jj) * 16
        acc = jnp.zeros((16,), jnp.int32)
        for l in range(16):
            acc = acc + plsc.load_gather(hist_v, [(base + lane) * 16 + l])
        total = jnp.sum(acc)
        pref = plsc.cumsum(acc)
        g = above + total - pref
        m = (g < kval) & (g + acc >= kval)
        b_sel = b_sel + jnp.sum(jnp.where(m, base + lane, 0))
        g_sel = g_sel + jnp.sum(jnp.where(m, g, 0))
        return (above + total, b_sel, g_sel)

    init = (jnp.int32(0), jnp.int32(0), jnp.int32(0))
    _, b_sel, g_sel = lax.fori_loop(0, 16, chunk_body, init)
    return b_sel, g_sel


def _sc_select_body(logits_hbm, outk_hbm, outi_hbm,
                    row_v, candk_v, candi_v, h0, outk_v, outi_v, cnts_v):
    wid = lax.axis_index("s") * 2 + lax.axis_index("c")
    lane = lax.iota(jnp.int32, 16)
    ones = jnp.ones((16,), jnp.int32)
    zeros_i = jnp.zeros((16,), jnp.int32)
    zeros_u = jnp.zeros((16,), jnp.uint32)

    def zero_hist(i, _):
        h0[pl.ds(i * 16, 16)] = zeros_i
        return 0

    def _popcnt(m):
        return plsc.all_reduce_population_count(m)[0]

    def row_body(r, _):
        row = wid * ROWS_PER_W + r
        pltpu.sync_copy(logits_hbm.at[row], row_v)

        @plsc.parallel_loop(0, 256, unroll=4)
        def _(i):
            h0[pl.ds(i * 16, 16)] = zeros_i

        @plsc.parallel_loop(0, V // 16, unroll=4)
        def _(i):
            key = _keys_of(row_v[pl.ds(i * 16, 16)])
            bin1 = (key >> 24).astype(jnp.int32)
            plsc.addupdate_scatter(h0, [bin1 * 16 + lane], ones)
        b1, g1 = _select_bucket(h0, jnp.int32(K), lane)
        b1u = b1.astype(jnp.uint32)

        cnts_v[pl.ds(16 * (V // 16 // 16), 16)] = zeros_i

        @plsc.parallel_loop(0, V // 16, unroll=4)
        def _(i):
            key = _keys_of(row_v[pl.ds(i * 16, 16)])
            m = (key >> 24) >= b1u
            pc = plsc.all_reduce_population_count(m)
            plsc.store_scatter(cnts_v, [lane * 0 + i], pc, mask=lane == 0)

        def prefix(j, carry):
            c = cnts_v[pl.ds(j * 16, 16)]
            total = jnp.sum(c)
            offs = carry + plsc.cumsum(c) - c
            cnts_v[pl.ds(j * 16, 16)] = offs
            return carry + total

        n1 = lax.fori_loop(0, V // 16 // 16 + 1, prefix, jnp.int32(0))

        @plsc.parallel_loop(0, V // 16, unroll=4)
        def _(i):
            key = _keys_of(row_v[pl.ds(i * 16, 16)])
            m = (key >> 24) >= b1u
            sp = jnp.minimum(cnts_v[pl.ds(i, 16)][0], CAP1)
            plsc.store_compressed(candk_v.at[pl.ds(sp, 16)], key, mask=m)
            plsc.store_compressed(candi_v.at[pl.ds(sp, 16)], i * 16 + lane, mask=m)

        n1 = jnp.minimum(n1, CAP1)
        ntrip = (n1 + 15) // 16

        lax.fori_loop(0, 256, zero_hist, 0)

        def pass_l2(i, _):
            kv = candk_v[pl.ds(i * 16, 16)]
            valid = lane < (n1 - i * 16)
            mb = valid & ((kv >> 24) == b1u)
            bin2 = ((kv >> 16) & jnp.uint32(0xFF)).astype(jnp.int32)
            plsc.addupdate_scatter(h0, [bin2 * 16 + lane], ones, mask=mb)
            return 0

        lax.fori_loop(0, ntrip, pass_l2, 0)
        b2, _ = _select_bucket(h0, jnp.int32(K) - g1, lane)
        t16 = (b1u << 8) | b2.astype(jnp.uint32)

        def zero_out(i, _):
            outk_v[pl.ds(i * 16, 16)] = zeros_u
            outi_v[pl.ds(i * 16, 16)] = zeros_i
            return 0

        lax.fori_loop(0, CAP2 // 16 + 1, zero_out, 0)

        def make_pass_c(cmp_eq):
            def pass_c(i, ptr):
                kv = candk_v[pl.ds(i * 16, 16)]
                iv = candi_v[pl.ds(i * 16, 16)]
                valid = lane < (n1 - i * 16)
                hi = kv >> 16
                m = valid & ((hi == t16) if cmp_eq else (hi > t16))
                cnt = _popcnt(m)
                sp = jnp.minimum(ptr, CAP2)
                plsc.store_compressed(outk_v.at[pl.ds(sp, 16)], kv, mask=m)
                plsc.store_compressed(outi_v.at[pl.ds(sp, 16)], iv, mask=m)
                return ptr + cnt

            return pass_c

        ng = lax.fori_loop(0, ntrip, make_pass_c(False), jnp.int32(0))
        lax.fori_loop(0, ntrip, make_pass_c(True), ng)

        pltpu.sync_copy(outk_v.at[pl.ds(0, CAP2)], outk_hbm.at[row])
        pltpu.sync_copy(outi_v.at[pl.ds(0, CAP2)], outi_hbm.at[row])
        return 0

    lax.fori_loop(0, ROWS_PER_W, row_body, 0)


def _tc_sample_body(keys_ref, idx_ref, tk_ref, tp_ref, temp_ref, rand_ref, out_ref):
    n = CAP2
    minint = jnp.int32(-2147483648)
    skey = lax.bitcast_convert_type(keys_ref[...], jnp.int32) ^ minint
    m = ~skey
    idx = idx_ref[...]
    col = lax.broadcasted_iota(jnp.int32, (B, n), 1)

    k = 2
    while k <= n:
        j = k // 2
        while j >= 1:
            lower = (col & j) == 0
            pm = jnp.where(lower, pltpu.roll(m, n - j, 1), pltpu.roll(m, j, 1))
            pidx = jnp.where(lower, pltpu.roll(idx, n - j, 1), pltpu.roll(idx, j, 1))
            precedes = (m < pm) | ((m == pm) & (idx < pidx))
            asc = (col & k) == 0
            keep_self = asc == (lower == precedes)
            m = jnp.where(keep_self, m, pm)
            idx = jnp.where(keep_self, idx, pidx)
            j //= 2
        k *= 2

    ki = (~m[:, :K]) ^ minint
    xbits = jnp.where(ki < 0, ki ^ minint, ~ki)
    vals = lax.bitcast_convert_type(xbits, jnp.float32)
    idx_s = idx[:, :K]
    colk = lax.broadcasted_iota(jnp.int32, (B, K), 1)

    tk = tk_ref[...]
    tp = tp_ref[...]
    temp = temp_ref[...]
    rand = rand_ref[...]

    vals = jnp.where(colk.astype(jnp.float32) >= tk, IGNORED, vals) / temp

    def softmax(v):
        mx = jnp.max(v, axis=1, keepdims=True)
        e = jnp.exp(v - mx)
        return e / jnp.sum(e, axis=1, keepdims=True)

    def cumsum(p):
        c = p
        d = 1
        while d < K:
            c = c + jnp.where(colk >= d, pltpu.roll(c, d, 1), 0.0)
            d *= 2
        return c

    csum = cumsum(softmax(vals))
    tp_eff = jnp.maximum(jnp.min(csum), tp)
    mask2 = (csum > tp_eff) & (colk >= 1)
    vals = jnp.where(mask2, IGNORED, vals)
    csum2 = cumsum(softmax(vals))
    counts = jnp.sum((rand > csum2).astype(jnp.int32), axis=1, keepdims=True)
    counts = jnp.minimum(counts, K - 1)
    out_ref[...] = jnp.sum(jnp.where(colk == counts, idx_s, 0), axis=1, keepdims=True)


def kernel(token_logits, sampling_params):
    mesh = plsc.VectorSubcoreMesh(core_axis_name="c", subcore_axis_name="s")
    sc_select = pl.kernel(
        _sc_select_body,
        out_type=(
            jax.ShapeDtypeStruct((B, CAP2), jnp.uint32),
            jax.ShapeDtypeStruct((B, CAP2), jnp.int32),
        ),
        mesh=mesh,
        compiler_params=pltpu.CompilerParams(needs_layout_passes=False),
        scratch_types=[
            pltpu.VMEM((V,), jnp.float32),
            pltpu.VMEM((CAP1 + 16,), jnp.uint32),
            pltpu.VMEM((CAP1 + 16,), jnp.int32),
            pltpu.VMEM((4096,), jnp.int32),
            pltpu.VMEM((CAP2 + 16,), jnp.uint32),
            pltpu.VMEM((CAP2 + 16,), jnp.int32),
            pltpu.VMEM((16 * (V // 16 // 16 + 2),), jnp.int32),
        ],
    )
    keys, idx = sc_select(token_logits)

    tk = sampling_params[:, 0:1]
    tp = sampling_params[:, 1:2]
    temp = sampling_params[:, 2:3]
    rand = jax.random.uniform(jax.random.key(1234), (B, 1), dtype=jnp.float32)

    tokens = pl.pallas_call(
        _tc_sample_body,
        out_shape=jax.ShapeDtypeStruct((B, 1), jnp.int32),
    )(keys, idx, tk, tp, temp, rand)
    return tokens.reshape(-1)
